# baseline (speedup 1.0000x reference)
import jax
import jax.numpy as jnp
from jax.experimental import pallas as pl
from jax.experimental.pallas import tpu as pltpu

_B = 16384
_IN_DIM = 768
_CLB_DIM = 32
_E_DIM = 64
_NUM_CODES = 1024
_NUM_LEVELS = 4
_BETA = 0.25
_TILE = 256

_PREC = jax.lax.Precision.DEFAULT


def _mlp(h, ws, bs):
    n = len(ws)
    for i in range(n):
        h = jnp.dot(h, ws[i][...], precision=_PREC) + bs[i][...]
        if i < n - 1:
            h = jax.nn.relu(h)
    return h


def _rq2(rx, ry, cbm2_ref, gat_ref, esq_ref, idx_ref, idx2_ref, loss_ref):
    t = _TILE
    accx = jnp.zeros_like(rx)
    accy = jnp.zeros_like(ry)
    lanes = jax.lax.broadcasted_iota(
        jnp.int32, (t, _NUM_CODES), 1).astype(jnp.float32)

    def _level(r, l):
        prod2 = jax.lax.dot_general(r, cbm2_ref[l], (((1,), (1,)), ((), ())),
                                    precision=_PREC)
        d = (jnp.sum(r * r, axis=1, keepdims=True) + prod2) + esq_ref[l]
        dmin = jnp.min(d, axis=1, keepdims=True)
        indf = jnp.min(jnp.where(d == dmin, lanes, jnp.inf),
                       axis=1, keepdims=True)
        oh = (lanes == indf).astype(jnp.float32)
        s = jnp.dot(oh, gat_ref[l], precision=_PREC)
        xq = (s[:, :_E_DIM] + s[:, _E_DIM:2 * _E_DIM]) + s[:, 2 * _E_DIM:]
        diff = xq - r
        sse = jnp.sum(diff * diff)
        return diff, sse, indf.astype(jnp.int32)

    for l in range(_NUM_LEVELS):
        diffx, ssex, indx = _level(rx, l)
        diffy, ssey, indy = _level(ry, l)
        loss_ref[0:1, l:l + 1, :] = jnp.full((1, 1, 128), ssex, jnp.float32)
        loss_ref[0:1, _NUM_LEVELS + l:_NUM_LEVELS + l + 1, :] = jnp.full(
            (1, 1, 128), ssey, jnp.float32)
        xq_st_x = rx + diffx
        rx = rx - xq_st_x
        accx = accx + xq_st_x
        xq_st_y = ry + diffy
        ry = ry - xq_st_y
        accy = accy + xq_st_y
        idx_ref[:, l:l + 1] = indx
        idx2_ref[:, l:l + 1] = indy
    return accx, accy


def _body(x_ref, y_ref,
          ew0, ew1, ew2, ew3, eb0, eb1, eb2, eb3,
          cw0, cw1, cw2, cw3, cb0, cb1, cb2, cb3,
          dw0, dw1, dw2, dw3, db0, db1, db2, db3,
          gw0, gw1, gw2, gw3, gb0, gb1, gb2, gb3,
          cbm2_ref, gat_ref, esq_ref,
          out_ref, out_clb_ref, xq_ref, yq_ref, idx_ref, idx2_ref, loss_ref):
    x_e = _mlp(x_ref[...], (ew0, ew1, ew2, ew3), (eb0, eb1, eb2, eb3))
    y_e = _mlp(y_ref[...], (cw0, cw1, cw2, cw3), (cb0, cb1, cb2, cb3))
    x_q, y_q = _rq2(x_e, y_e, cbm2_ref,
                    gat_ref, esq_ref, idx_ref, idx2_ref, loss_ref)
    xq_ref[...] = x_q
    yq_ref[...] = y_q
    out_ref[...] = _mlp(x_q, (dw0, dw1, dw2, dw3), (db0, db1, db2, db3))
    out_clb_ref[...] = _mlp(y_e, (gw0, gw1, gw2, gw3), (gb0, gb1, gb2, gb3))


def kernel(x, y, labels, labels_2, enc_params, clb_enc_params, dec_params,
           clb_dec_params, codebooks):
    del labels, labels_2
    nt = _B // _TILE
    cb = jnp.stack(codebooks)
    cbm2 = -2.0 * cb
    hi = jax.lax.reduce_precision(cb, 8, 7)
    rem = cb - hi
    mid = jax.lax.reduce_precision(rem, 8, 7)
    lo = rem - mid
    gat = jnp.concatenate([hi, mid, lo], axis=2)
    esq = jnp.stack([jnp.sum(e ** 2, axis=1)[None, :] for e in codebooks])

    def _wb(ps):
        ws = [ps[2 * i] for i in range(4)]
        bs = [ps[2 * i + 1].reshape(1, -1) for i in range(4)]
        return ws, bs

    ews, ebs = _wb(enc_params)
    cws, cbs = _wb(clb_enc_params)
    dws, dbs = _wb(dec_params)
    gws, gbs = _wb(clb_dec_params)

    def _full(a):
        nd = a.ndim
        return pl.BlockSpec(a.shape, lambda i, _n=nd: (0,) * _n)

    row = lambda d: pl.BlockSpec((_TILE, d), lambda i: (i, 0))
    weights = (ews + ebs + cws + cbs + dws + dbs + gws + gbs
               + [cbm2, gat, esq])

    out_shapes = (
        jax.ShapeDtypeStruct((_B, _IN_DIM), jnp.float32),
        jax.ShapeDtypeStruct((_B, _CLB_DIM), jnp.float32),
        jax.ShapeDtypeStruct((_B, _E_DIM), jnp.float32),
        jax.ShapeDtypeStruct((_B, _E_DIM), jnp.float32),
        jax.ShapeDtypeStruct((_B, _NUM_LEVELS), jnp.int32),
        jax.ShapeDtypeStruct((_B, _NUM_LEVELS), jnp.int32),
        jax.ShapeDtypeStruct((nt, 2 * _NUM_LEVELS, 128), jnp.float32),
    )
    out_specs = (
        row(_IN_DIM), row(_CLB_DIM), row(_E_DIM), row(_E_DIM),
        row(_NUM_LEVELS), row(_NUM_LEVELS),
        pl.BlockSpec((1, 2 * _NUM_LEVELS, 128), lambda i: (i, 0, 0)),
    )

    out, out_clb, x_q, y_q, indices, indices_2, loss_parts = pl.pallas_call(
        _body,
        grid=(nt,),
        in_specs=[row(_IN_DIM), row(_CLB_DIM)] + [_full(w) for w in weights],
        out_specs=out_specs,
        out_shape=out_shapes,
        compiler_params=pltpu.CompilerParams(
            dimension_semantics=("parallel",)),
    )(x, y, *weights)

    sse = jnp.sum(loss_parts[:, :, 0], axis=0)
    m = sse / jnp.float32(_B * _E_DIM)
    per_level = m + _BETA * m
    rq_loss = jnp.mean(per_level[:_NUM_LEVELS])
    rq_loss_2 = jnp.mean(per_level[_NUM_LEVELS:])
    return (out, out_clb, rq_loss, rq_loss_2, indices, indices_2, x_q, y_q)


# speedup vs baseline: 1.1742x; 1.1742x over previous
import jax
import jax.numpy as jnp
from jax.experimental import pallas as pl
from jax.experimental.pallas import tpu as pltpu

_B = 16384
_IN_DIM = 768
_CLB_DIM = 32
_E_DIM = 64
_NUM_CODES = 1024
_NUM_LEVELS = 4
_BETA = 0.25
_TILE = 512

_PREC = jax.lax.Precision.DEFAULT


def _mlp(h, ws, bs):
    n = len(ws)
    for i in range(n):
        h = jnp.dot(h, ws[i][...], precision=_PREC) + bs[i][...]
        if i < n - 1:
            h = jax.nn.relu(h)
    return h


def _rq2(rx, ry, cbm2_ref, gat_ref, esq_ref, idx_ref, idx2_ref, loss_ref):
    t = _TILE
    accx = jnp.zeros_like(rx)
    accy = jnp.zeros_like(ry)
    lanes = jax.lax.broadcasted_iota(
        jnp.int32, (t, _NUM_CODES), 1).astype(jnp.float32)

    def _level(r, l):
        prod2 = jax.lax.dot_general(r, cbm2_ref[l], (((1,), (1,)), ((), ())),
                                    precision=_PREC)
        d = (jnp.sum(r * r, axis=1, keepdims=True) + prod2) + esq_ref[l]
        dmin = jnp.min(d, axis=1, keepdims=True)
        indf = jnp.min(jnp.where(d == dmin, lanes, jnp.inf),
                       axis=1, keepdims=True)
        oh = (lanes == indf).astype(jnp.float32)
        s = jnp.dot(oh, gat_ref[l], precision=_PREC)
        xq = (s[:, :_E_DIM] + s[:, _E_DIM:2 * _E_DIM]) + s[:, 2 * _E_DIM:]
        diff = xq - r
        sse = jnp.sum(diff * diff)
        return diff, sse, indf.astype(jnp.int32)

    for l in range(_NUM_LEVELS):
        diffx, ssex, indx = _level(rx, l)
        diffy, ssey, indy = _level(ry, l)
        loss_ref[0:1, l:l + 1, :] = jnp.full((1, 1, 128), ssex, jnp.float32)
        loss_ref[0:1, _NUM_LEVELS + l:_NUM_LEVELS + l + 1, :] = jnp.full(
            (1, 1, 128), ssey, jnp.float32)
        xq_st_x = rx + diffx
        rx = rx - xq_st_x
        accx = accx + xq_st_x
        xq_st_y = ry + diffy
        ry = ry - xq_st_y
        accy = accy + xq_st_y
        idx_ref[:, l:l + 1] = indx
        idx2_ref[:, l:l + 1] = indy
    return accx, accy


def _body(x_ref, y_ref,
          ew0, ew1, ew2, ew3, eb0, eb1, eb2, eb3,
          cw0, cw1, cw2, cw3, cb0, cb1, cb2, cb3,
          dw0, dw1, dw2, dw3, db0, db1, db2, db3,
          gw0, gw1, gw2, gw3, gb0, gb1, gb2, gb3,
          cbm2_ref, gat_ref, esq_ref,
          out_ref, out_clb_ref, xq_ref, yq_ref, idx_ref, idx2_ref, loss_ref):
    x_e = _mlp(x_ref[...], (ew0, ew1, ew2, ew3), (eb0, eb1, eb2, eb3))
    y_e = _mlp(y_ref[...], (cw0, cw1, cw2, cw3), (cb0, cb1, cb2, cb3))
    x_q, y_q = _rq2(x_e, y_e, cbm2_ref,
                    gat_ref, esq_ref, idx_ref, idx2_ref, loss_ref)
    xq_ref[...] = x_q
    yq_ref[...] = y_q
    out_ref[...] = _mlp(x_q, (dw0, dw1, dw2, dw3), (db0, db1, db2, db3))
    out_clb_ref[...] = _mlp(y_e, (gw0, gw1, gw2, gw3), (gb0, gb1, gb2, gb3))


def kernel(x, y, labels, labels_2, enc_params, clb_enc_params, dec_params,
           clb_dec_params, codebooks):
    del labels, labels_2
    nt = _B // _TILE
    cb = jnp.stack(codebooks)
    cbm2 = -2.0 * cb
    hi = jax.lax.reduce_precision(cb, 8, 7)
    rem = cb - hi
    mid = jax.lax.reduce_precision(rem, 8, 7)
    lo = rem - mid
    gat = jnp.concatenate([hi, mid, lo], axis=2)
    esq = jnp.stack([jnp.sum(e ** 2, axis=1)[None, :] for e in codebooks])

    def _wb(ps):
        ws = [ps[2 * i] for i in range(4)]
        bs = [ps[2 * i + 1].reshape(1, -1) for i in range(4)]
        return ws, bs

    ews, ebs = _wb(enc_params)
    cws, cbs = _wb(clb_enc_params)
    dws, dbs = _wb(dec_params)
    gws, gbs = _wb(clb_dec_params)

    def _full(a):
        nd = a.ndim
        return pl.BlockSpec(a.shape, lambda i, _n=nd: (0,) * _n)

    row = lambda d: pl.BlockSpec((_TILE, d), lambda i: (i, 0))
    weights = (ews + ebs + cws + cbs + dws + dbs + gws + gbs
               + [cbm2, gat, esq])

    out_shapes = (
        jax.ShapeDtypeStruct((_B, _IN_DIM), jnp.float32),
        jax.ShapeDtypeStruct((_B, _CLB_DIM), jnp.float32),
        jax.ShapeDtypeStruct((_B, _E_DIM), jnp.float32),
        jax.ShapeDtypeStruct((_B, _E_DIM), jnp.float32),
        jax.ShapeDtypeStruct((_B, _NUM_LEVELS), jnp.int32),
        jax.ShapeDtypeStruct((_B, _NUM_LEVELS), jnp.int32),
        jax.ShapeDtypeStruct((nt, 2 * _NUM_LEVELS, 128), jnp.float32),
    )
    out_specs = (
        row(_IN_DIM), row(_CLB_DIM), row(_E_DIM), row(_E_DIM),
        row(_NUM_LEVELS), row(_NUM_LEVELS),
        pl.BlockSpec((1, 2 * _NUM_LEVELS, 128), lambda i: (i, 0, 0)),
    )

    out, out_clb, x_q, y_q, indices, indices_2, loss_parts = pl.pallas_call(
        _body,
        grid=(nt,),
        in_specs=[row(_IN_DIM), row(_CLB_DIM)] + [_full(w) for w in weights],
        out_specs=out_specs,
        out_shape=out_shapes,
        compiler_params=pltpu.CompilerParams(
            dimension_semantics=("parallel",)),
    )(x, y, *weights)

    sse = jnp.sum(loss_parts[:, :, 0], axis=0)
    m = sse / jnp.float32(_B * _E_DIM)
    per_level = m + _BETA * m
    rq_loss = jnp.mean(per_level[:_NUM_LEVELS])
    rq_loss_2 = jnp.mean(per_level[_NUM_LEVELS:])
    return (out, out_clb, rq_loss, rq_loss_2, indices, indices_2, x_q, y_q)
